# baseline (speedup 1.0000x reference)
import functools

import jax
import jax.numpy as jnp
from jax import lax
from jax.experimental import pallas as pl
from jax.experimental.pallas import tpu as pltpu
from jax.experimental.pallas import tpu_sc as plsc

B, C, H, W = 4, 96, 384, 384
HW = H * W
K = HW // 2
H_BLK = 32
N_HBLK = H // H_BLK
H_BLK1 = 32
N_HBLK1 = H // H_BLK1


def _importance_body(a_ref, b_ref, s_ref):
    a = a_ref[0]
    b = b_ref[0]
    s_ref[0, 0] = jnp.sum(jnp.abs(a), axis=0)
    s_ref[1, 0] = jnp.sum(jnp.abs(b), axis=0)
    s_ref[2, 0] = jnp.sum(jnp.abs(a + b), axis=0)


def _importance(fa, fb):
    return pl.pallas_call(
        _importance_body,
        grid=(B, N_HBLK1),
        in_specs=[
            pl.BlockSpec((1, C, H_BLK1, W), lambda b, h: (b, 0, h, 0)),
            pl.BlockSpec((1, C, H_BLK1, W), lambda b, h: (b, 0, h, 0)),
        ],
        out_specs=pl.BlockSpec((3, 1, H_BLK1, W), lambda b, h: (0, b, h, 0)),
        out_shape=jax.ShapeDtypeStruct((3, B, H, W), jnp.float32),
    )(fa, fb)


def _threshold_body(s_ref, t_ref):
    def body(_, carry):
        lo, hi = carry
        mid = lo + (hi - lo) // 2
        mid_f = jax.lax.bitcast_convert_type(mid, jnp.float32)
        cnt = jnp.sum((s_ref[...] > mid_f).astype(jnp.int32), axis=1,
                      keepdims=True)
        take_hi = cnt < K
        return (jnp.where(take_hi, lo, mid), jnp.where(take_hi, mid, hi))

    lo0 = jnp.full((3 * B, 1), -1, jnp.int32)
    hi0 = jnp.full((3 * B, 1), 0x7F800000, jnp.int32)
    _, hi = jax.lax.fori_loop(0, 31, body, (lo0, hi0))
    t_ref[...] = jnp.broadcast_to(
        jax.lax.bitcast_convert_type(hi, jnp.float32), (3 * B, 128))


def _thresholds(s):
    s2 = s.reshape(3 * B, HW)
    t = pl.pallas_call(
        _threshold_body,
        out_shape=jax.ShapeDtypeStruct((3 * B, 128), jnp.float32),
    )(s2)
    return t[:, 0].reshape(3, B)


NMAPS = 3 * B
MPC = NMAPS // 2
EPW = HW // 16
NV = EPW // 16
_SHIFTS = (24, 16, 8, 0)


def _sc_threshold_body(maps_hbm, out_hbm, buf, hist, acc, tmp, ovec, parts):
    c = lax.axis_index("c")
    s = lax.axis_index("s")
    zero = jnp.zeros((16,), jnp.int32)
    ones = jnp.ones((16,), jnp.int32)
    iota16 = lax.iota(jnp.int32, 16)

    for m in range(MPC):
        pltpu.sync_copy(maps_hbm.at[c * MPC + m, pl.ds(s * EPW, EPW)],
                        buf.at[m])

    kks = [jnp.int32(K)] * MPC
    prefixes = [jnp.int32(0)] * MPC

    for r, shift in enumerate(_SHIFTS):
        def zbody(i, _):
            hist[pl.ds(i * 16, 16)] = zero
            return 0
        lax.fori_loop(0, MPC * 16, zbody, 0, unroll=8)

        for m in range(MPC):
            base = jnp.int32(m * 256)
            if r == 0:
                def pbody(j, _, m=m, base=base):
                    bits = jax.lax.bitcast_convert_type(
                        buf[m, pl.ds(j * 16, 16)], jnp.int32)
                    byte = jnp.right_shift(bits, shift) & 255
                    plsc.addupdate_scatter(hist, [byte + base], ones)
                    return 0
            else:
                hi_shift = shift + 8
                pref_hi = jnp.right_shift(prefixes[m], hi_shift)
                def pbody(j, _, m=m, base=base, hi_shift=hi_shift,
                          pref_hi=pref_hi):
                    bits = jax.lax.bitcast_convert_type(
                        buf[m, pl.ds(j * 16, 16)], jnp.int32)
                    byte = jnp.right_shift(bits, shift) & 255
                    act = jnp.right_shift(bits, hi_shift) == pref_hi
                    plsc.addupdate_scatter(hist, [byte + base], ones, mask=act)
                    return 0
            lax.fori_loop(0, NV, pbody, 0, unroll=4)

        if r > 0:
            plsc.subcore_barrier()
        pltpu.sync_copy(hist, parts.at[s])
        plsc.subcore_barrier()
        def zacc(i, _):
            acc[pl.ds(i * 16, 16)] = zero
            return 0
        lax.fori_loop(0, MPC * 16, zacc, 0, unroll=8)
        for t in range(16):
            pltpu.sync_copy(parts.at[t], tmp)
            def abody(i, _):
                blk = pl.ds(i * 16, 16)
                acc[blk] = acc[blk] + tmp[blk]
                return 0
            lax.fori_loop(0, MPC * 16, abody, 0, unroll=8)

        for m in range(MPC):
            kk = kks[m]
            def gbody(i, st, m=m):
                carry, v_found, above, done = st
                g = 15 - i
                vg = acc[pl.ds(m * 256 + g * 16, 16)]
                rv = lax.rev(vg, (0,))
                rc = plsc.cumsum(rv)
                tot = rc + carry
                cross = tot >= kk
                jstar = jnp.min(jnp.where(cross, iota16, jnp.int32(16)))
                hit = jnp.logical_and(done == 0, jstar < 16)
                sel = iota16 == jstar
                e_rc = jnp.sum(jnp.where(sel, rc, 0))
                e_rv = jnp.sum(jnp.where(sel, rv, 0))
                v_found = jnp.where(hit, (g * 16 + 15) - jstar, v_found)
                above = jnp.where(hit, carry + e_rc - e_rv, above)
                done = jnp.where(jstar < 16, jnp.int32(1), done)
                return (carry + jnp.sum(vg), v_found, above, done)
            _, v_found, above, _ = lax.fori_loop(
                0, 16, gbody,
                (jnp.int32(0), jnp.int32(0), jnp.int32(0), jnp.int32(0)))
            kks[m] = kk - above
            prefixes[m] = prefixes[m] | (v_found << shift)

    pref_mine = prefixes[0]
    for m in range(1, MPC):
        pref_mine = jnp.where(s == m, prefixes[m], pref_mine)
    ovec[...] = jax.lax.bitcast_convert_type(
        jnp.broadcast_to(pref_mine, (16,)), jnp.float32)
    @pl.when(s < MPC)
    def _():
        pltpu.sync_copy(ovec, out_hbm.at[c * MPC + s])


def _sc_thresholds(sm):
    mesh = plsc.VectorSubcoreMesh(core_axis_name="c", subcore_axis_name="s")
    t = pl.kernel(
        _sc_threshold_body,
        mesh=mesh,
        out_type=jax.ShapeDtypeStruct((NMAPS, 16), jnp.float32),
        scratch_types=[
            pltpu.VMEM((MPC, EPW), jnp.float32),
            pltpu.VMEM((MPC * 256,), jnp.int32),
            pltpu.VMEM((MPC * 256,), jnp.int32),
            pltpu.VMEM((MPC * 256,), jnp.int32),
            pltpu.VMEM((16,), jnp.float32),
            pltpu.VMEM_SHARED((16, MPC * 256), jnp.int32),
        ],
        compiler_params=pltpu.CompilerParams(needs_layout_passes=False),
    )(sm.reshape(NMAPS, HW))
    return t[:, 0].reshape(3, B)


def _apply_body(t_ref, a_ref, b_ref, s_ref, o1_ref, o2_ref, o3_ref):
    b = pl.program_id(0)
    m1 = (s_ref[0] >= t_ref[0, b]).astype(jnp.float32)
    m2 = (s_ref[1] >= t_ref[1, b]).astype(jnp.float32)
    m3 = (s_ref[2] >= t_ref[2, b]).astype(jnp.float32)
    fa = a_ref[0]
    fb = b_ref[0]
    o1_ref[0] = fa * m1
    o2_ref[0] = fb * m2
    o3_ref[0] = (0.5 * (fa + fb)) * m3


def _apply(t, fa, fb, s):
    shp = jax.ShapeDtypeStruct((B, C, H, W), jnp.float32)
    return pl.pallas_call(
        _apply_body,
        grid=(B, N_HBLK),
        in_specs=[
            pl.BlockSpec(memory_space=pltpu.SMEM),
            pl.BlockSpec((1, C, H_BLK, W), lambda b, h: (b, 0, h, 0)),
            pl.BlockSpec((1, C, H_BLK, W), lambda b, h: (b, 0, h, 0)),
            pl.BlockSpec((3, 1, H_BLK, W), lambda b, h: (0, b, h, 0)),
        ],
        out_specs=[
            pl.BlockSpec((1, C, H_BLK, W), lambda b, h: (b, 0, h, 0)),
            pl.BlockSpec((1, C, H_BLK, W), lambda b, h: (b, 0, h, 0)),
            pl.BlockSpec((1, C, H_BLK, W), lambda b, h: (b, 0, h, 0)),
        ],
        out_shape=(shp, shp, shp),
    )(t, fa, fb, s)


def kernel(F_self, F_others):
    s = _importance(F_self, F_others)
    t = _sc_thresholds(s)
    sel1, sel2, sel3 = _apply(t, F_self, F_others, s)
    return (sel1, sel2, sel3)
